# baseline (speedup 1.0000x reference)
import functools
import math

import jax
import jax.numpy as jnp
from jax import lax
from jax.experimental import pallas as pl
from jax.experimental.pallas import tpu as pltpu
from jax.experimental.pallas import tpu_sc as plsc

N = 10000
E = 320000
D = 128
FC0 = 16
FC1 = 64
NUM_NEIGHBORS = 32.0

NC = 2
NS = 16
NW = NC * NS
CH = 40
NP = 10240
RPT = NP // NS

NSPLIT = 5
ES = E // NSPLIT
EPT = ES // NW
NCHUNK = EPT // CH
BE = 4000
NBLK = ES // BE


def _edge_fc_body(elem_ref, eattr_ref, wfc1_ref, wfc2_ref, out_ref):
    h = jnp.dot(elem_ref[...], wfc1_ref[...], preferred_element_type=jnp.float32)
    h = h * (1.0 / math.sqrt(float(FC0)))
    h = h * jax.nn.sigmoid(h)
    w = jnp.dot(h, wfc2_ref[...], preferred_element_type=jnp.float32)
    scale = (1.0 / math.sqrt(float(FC1))) * (1.0 / math.sqrt(NUM_NEIGHBORS))
    out_ref[...] = w * eattr_ref[...] * scale


def _make_edge_fc(k):
    return pl.pallas_call(
        _edge_fc_body,
        grid=(NBLK,),
        in_specs=[
            pl.BlockSpec((BE, FC0), lambda i: (k * NBLK + i, 0)),
            pl.BlockSpec((BE, 1), lambda i: (k * NBLK + i, 0)),
            pl.BlockSpec((FC0, FC1), lambda i: (0, 0)),
            pl.BlockSpec((FC1, D), lambda i: (0, 0)),
        ],
        out_specs=pl.BlockSpec((BE, D), lambda i: (i, 0)),
        out_shape=jax.ShapeDtypeStruct((ES, D), jnp.float32),
    )


def _node_lin_body(ni_ref, na_ref, w1_ref, out_ref):
    x = jnp.dot(ni_ref[...], w1_ref[...], preferred_element_type=jnp.float32)
    out_ref[...] = x * na_ref[...] * (1.0 / math.sqrt(float(D)))


_node_lin = pl.pallas_call(
    _node_lin_body,
    out_shape=jax.ShapeDtypeStruct((N, D), jnp.float32),
)


def _make_sc_body(k, first):
    def _sc_body(x_hbm, w_hbm, src_hbm, dst_hbm, init_hbm, out_hbm,
                 src0_v, src1_v, dst0_v, dst1_v,
                 xr0_v, xr1_v, wr0_v, wr1_v, acc_sh,
                 sem_g0, sem_g1, sem_w0, sem_w1, sem_i0, sem_i1):
        c = lax.axis_index("c")
        s = lax.axis_index("s")
        wid = s * NC + c
        row0 = s * RPT
        if first:
            pltpu.sync_copy(init_hbm.at[pl.ds(row0, RPT)],
                            acc_sh.at[pl.ds(row0, RPT)])
        else:
            pltpu.sync_copy(init_hbm.at[pl.ds(c * NP + row0, RPT)],
                            acc_sh.at[pl.ds(row0, RPT)])
        plsc.subcore_barrier()

        src = (src0_v, src1_v)
        dst = (dst0_v, dst1_v)
        xr = (xr0_v, xr1_v)
        wr = (wr0_v, wr1_v)
        sem_g = (sem_g0, sem_g1)
        sem_w = (sem_w0, sem_w1)
        sem_i = (sem_i0, sem_i1)
        ibase = k * ES + wid * EPT
        wbase = wid * EPT

        def idx_start(i, b):
            @pl.when(i < NCHUNK)
            def _():
                base = ibase + jnp.minimum(i, NCHUNK - 1) * CH
                pltpu.async_copy(src_hbm.at[pl.ds(base, CH)], src[b], sem_i[b])
                pltpu.async_copy(dst_hbm.at[pl.ds(base, CH)], dst[b], sem_i[b])

        def idx_wait(i, b):
            @pl.when(i < NCHUNK)
            def _():
                pltpu.make_async_copy(
                    src_hbm.at[pl.ds(0, CH)], src[b], sem_i[b]).wait()
                pltpu.make_async_copy(
                    dst_hbm.at[pl.ds(0, CH)], dst[b], sem_i[b]).wait()

        def data_start(i, b):
            @pl.when(i < NCHUNK)
            def _():
                base = wbase + jnp.minimum(i, NCHUNK - 1) * CH
                pltpu.async_copy(w_hbm.at[pl.ds(base, CH)], wr[b], sem_w[b])
                pltpu.async_copy(x_hbm.at[src[b]], xr[b], sem_g[b])

        def data_wait(b):
            pltpu.make_async_copy(w_hbm.at[pl.ds(0, CH)], wr[b], sem_w[b]).wait()
            pltpu.make_async_copy(x_hbm.at[pl.ds(0, CH)], xr[b], sem_g[b]).wait()

        idx_start(0, 0)
        idx_wait(0, 0)
        data_start(0, 0)
        idx_start(1, 1)

        def step(i, b):
            data_wait(b)

            def mrow(r, carry2):
                for cc in range(D // 16):
                    sl = pl.ds(cc * 16, 16)
                    xr[b][r, sl] = xr[b][r, sl] * wr[b][r, sl]
                return carry2

            lax.fori_loop(0, CH, mrow, 0)
            idx_wait(i + 1, 1 - b)
            data_start(i + 1, 1 - b)
            pltpu.sync_copy(xr[b], acc_sh.at[dst[b]], add=True)
            idx_start(i + 2, b)

        def pair(j, carry):
            step(2 * j, 0)
            step(2 * j + 1, 1)
            return carry

        lax.fori_loop(0, NCHUNK // 2, pair, 0)

        plsc.subcore_barrier()
        pltpu.sync_copy(acc_sh.at[pl.ds(row0, RPT)],
                        out_hbm.at[pl.ds(c * NP + row0, RPT)])

    return _sc_body


def _make_sc(k):
    return functools.partial(
        pl.kernel,
        out_type=jax.ShapeDtypeStruct((NC * NP, D), jnp.float32),
        mesh=plsc.VectorSubcoreMesh(core_axis_name="c", subcore_axis_name="s"),
        scratch_types=[
            pltpu.VMEM((CH,), jnp.int32),
            pltpu.VMEM((CH,), jnp.int32),
            pltpu.VMEM((CH,), jnp.int32),
            pltpu.VMEM((CH,), jnp.int32),
            pltpu.VMEM((CH, D), jnp.float32),
            pltpu.VMEM((CH, D), jnp.float32),
            pltpu.VMEM((CH, D), jnp.float32),
            pltpu.VMEM((CH, D), jnp.float32),
            pltpu.VMEM_SHARED((NP, D), jnp.float32),
            pltpu.SemaphoreType.DMA,
            pltpu.SemaphoreType.DMA,
            pltpu.SemaphoreType.DMA,
            pltpu.SemaphoreType.DMA,
            pltpu.SemaphoreType.DMA,
            pltpu.SemaphoreType.DMA,
        ],
    )(_make_sc_body(k, first=(k == 0)))


_sc_calls = [_make_sc(k) for k in range(NSPLIT)]


def _final_body(ni_ref, na_ref, part_ref, wsc_ref, w2_ref, out_ref):
    agg = part_ref[0:N, :] + part_ref[NP:NP + N, :]
    s = jnp.dot(ni_ref[...], wsc_ref[...], preferred_element_type=jnp.float32)
    xo = jnp.dot(agg, w2_ref[...], preferred_element_type=jnp.float32)
    c_s = math.sin(math.pi / 8.0) / math.sqrt(float(D))
    c_x = math.cos(math.pi / 8.0) / math.sqrt(float(D))
    out_ref[...] = (s * c_s + xo * c_x) * na_ref[...]


_final = pl.pallas_call(
    _final_body,
    out_shape=jax.ShapeDtypeStruct((N, D), jnp.float32),
)


def kernel(node_input, node_attr, edge_src, edge_dst, edge_attr,
           edge_length_embedded, W_sc, W_lin1, W_fc1, W_fc2, W_lin2):
    w_slices = [
        _make_edge_fc(k)(edge_length_embedded, edge_attr, W_fc1, W_fc2)
        for k in range(NSPLIT)
    ]
    x = _node_lin(node_input, node_attr, W_lin1[:, 0, :])
    acc = jnp.zeros((NP, D), dtype=jnp.float32)
    for k in range(NSPLIT):
        acc = _sc_calls[k](x, w_slices[k], edge_src, edge_dst, acc)
    return _final(node_input, node_attr, acc, W_sc[:, 0, :], W_lin2[:, 0, :])


# speedup vs baseline: 1.2043x; 1.2043x over previous
import functools
import math

import jax
import jax.numpy as jnp
from jax import lax
from jax.experimental import pallas as pl
from jax.experimental.pallas import tpu as pltpu
from jax.experimental.pallas import tpu_sc as plsc

N = 10000
E = 320000
D = 128
FC0 = 16
FC1 = 64
NUM_NEIGHBORS = 32.0

NC = 2
NS = 16
NW = NC * NS
CH = 40
NP = 10240
RPT = NP // NS

NSPLIT = 5
ES = E // NSPLIT
EPT = ES // NW
NCHUNK = EPT // CH
BE = 4000
NBLK = ES // BE


def _edge_fc_body(elem_ref, eattr_ref, wfc1_ref, wfc2_ref, out_ref):
    h = jnp.dot(elem_ref[...], wfc1_ref[...], preferred_element_type=jnp.float32)
    h = h * (1.0 / math.sqrt(float(FC0)))
    h = h * jax.nn.sigmoid(h)
    w = jnp.dot(h, wfc2_ref[...], preferred_element_type=jnp.float32)
    scale = (1.0 / math.sqrt(float(FC1))) * (1.0 / math.sqrt(NUM_NEIGHBORS))
    out_ref[...] = w * eattr_ref[...] * scale


def _make_edge_fc(k):
    return pl.pallas_call(
        _edge_fc_body,
        grid=(NBLK,),
        in_specs=[
            pl.BlockSpec((BE, FC0), lambda i: (k * NBLK + i, 0)),
            pl.BlockSpec((BE, 1), lambda i: (k * NBLK + i, 0)),
            pl.BlockSpec((FC0, FC1), lambda i: (0, 0)),
            pl.BlockSpec((FC1, D), lambda i: (0, 0)),
        ],
        out_specs=pl.BlockSpec((BE, D), lambda i: (i, 0)),
        out_shape=jax.ShapeDtypeStruct((ES, D), jnp.float32),
    )


def _node_lin_body(ni_ref, na_ref, w1_ref, out_ref):
    x = jnp.dot(ni_ref[...], w1_ref[...], preferred_element_type=jnp.float32)
    out_ref[...] = x * na_ref[...] * (1.0 / math.sqrt(float(D)))


_node_lin = pl.pallas_call(
    _node_lin_body,
    out_shape=jax.ShapeDtypeStruct((N, D), jnp.float32),
)


def _make_sc_body(k, first):
    def _sc_body(x_hbm, w_hbm, src_hbm, dst_hbm, init_hbm, out_hbm,
                 src0_v, src1_v, sdst0_v, sdst1_v,
                 xr0_v, xr1_v, wr0_v, wr1_v, acc_sh,
                 sem_g0, sem_g1, sem_w0, sem_w1, sem_i0, sem_i1,
                 sem_s0, sem_s1, sem_z):
        c = lax.axis_index("c")
        s = lax.axis_index("s")
        wid = s * NC + c
        row0 = s * RPT
        if first:
            seed = pltpu.async_copy(init_hbm.at[pl.ds(row0, RPT)],
                                    acc_sh.at[pl.ds(row0, RPT)], sem_z)
        else:
            seed = pltpu.async_copy(init_hbm.at[pl.ds(c * NP + row0, RPT)],
                                    acc_sh.at[pl.ds(row0, RPT)], sem_z)

        src = (src0_v, src1_v)
        sdst = (sdst0_v, sdst1_v)
        xr = (xr0_v, xr1_v)
        wr = (wr0_v, wr1_v)
        sem_g = (sem_g0, sem_g1)
        sem_w = (sem_w0, sem_w1)
        sem_i = (sem_i0, sem_i1)
        sem_s = (sem_s0, sem_s1)
        ibase = k * ES + wid * EPT
        wbase = wid * EPT

        def idx_start(i, b):
            @pl.when(i < NCHUNK)
            def _():
                base = ibase + jnp.minimum(i, NCHUNK - 1) * CH
                pltpu.async_copy(src_hbm.at[pl.ds(base, CH)], src[b], sem_i[b])

        def idx_wait(i, b):
            @pl.when(i < NCHUNK)
            def _():
                pltpu.make_async_copy(
                    src_hbm.at[pl.ds(0, CH)], src[b], sem_i[b]).wait()

        def data_start(i, b):
            @pl.when(i < NCHUNK)
            def _():
                base = wbase + jnp.minimum(i, NCHUNK - 1) * CH
                dbase = ibase + jnp.minimum(i, NCHUNK - 1) * CH
                pltpu.async_copy(w_hbm.at[pl.ds(base, CH)], wr[b], sem_w[b])
                pltpu.async_copy(dst_hbm.at[pl.ds(dbase, CH)], sdst[b], sem_w[b])
                pltpu.async_copy(x_hbm.at[src[b]], xr[b], sem_g[b])

        def data_wait(b):
            pltpu.make_async_copy(w_hbm.at[pl.ds(0, CH)], wr[b], sem_w[b]).wait()
            pltpu.make_async_copy(dst_hbm.at[pl.ds(0, CH)], sdst[b], sem_w[b]).wait()
            pltpu.make_async_copy(x_hbm.at[pl.ds(0, CH)], xr[b], sem_g[b]).wait()

        def scatter_wait(i, b):
            @pl.when(i >= 0)
            def _():
                pltpu.make_async_copy(xr[b], acc_sh.at[sdst[b]], sem_s[b]).wait()

        idx_start(0, 0)
        idx_wait(0, 0)
        data_start(0, 0)
        idx_start(1, 1)
        seed.wait()
        plsc.subcore_barrier()

        def step(i, b):
            scatter_wait(i - 1, 1 - b)
            idx_wait(i + 1, 1 - b)
            data_start(i + 1, 1 - b)
            data_wait(b)
            idx_start(i + 2, b)

            def mrow(r, carry2):
                for cc in range(D // 16):
                    sl = pl.ds(cc * 16, 16)
                    xr[b][r, sl] = xr[b][r, sl] * wr[b][r, sl]
                return carry2

            lax.fori_loop(0, CH, mrow, 0)
            pltpu.async_copy(xr[b], acc_sh.at[sdst[b]], sem_s[b], add=True)

        def pair(j, carry):
            step(2 * j, 0)
            step(2 * j + 1, 1)
            return carry

        lax.fori_loop(0, NCHUNK // 2, pair, 0)
        scatter_wait(NCHUNK - 1, 1)

        plsc.subcore_barrier()
        pltpu.sync_copy(acc_sh.at[pl.ds(row0, RPT)],
                        out_hbm.at[pl.ds(c * NP + row0, RPT)])

    return _sc_body


def _make_sc(k):
    return functools.partial(
        pl.kernel,
        out_type=jax.ShapeDtypeStruct((NC * NP, D), jnp.float32),
        mesh=plsc.VectorSubcoreMesh(core_axis_name="c", subcore_axis_name="s"),
        scratch_types=[
            pltpu.VMEM((CH,), jnp.int32),
            pltpu.VMEM((CH,), jnp.int32),
            pltpu.VMEM((CH,), jnp.int32),
            pltpu.VMEM((CH,), jnp.int32),
            pltpu.VMEM((CH, D), jnp.float32),
            pltpu.VMEM((CH, D), jnp.float32),
            pltpu.VMEM((CH, D), jnp.float32),
            pltpu.VMEM((CH, D), jnp.float32),
            pltpu.VMEM_SHARED((NP, D), jnp.float32),
            pltpu.SemaphoreType.DMA,
            pltpu.SemaphoreType.DMA,
            pltpu.SemaphoreType.DMA,
            pltpu.SemaphoreType.DMA,
            pltpu.SemaphoreType.DMA,
            pltpu.SemaphoreType.DMA,
            pltpu.SemaphoreType.DMA,
            pltpu.SemaphoreType.DMA,
            pltpu.SemaphoreType.DMA,
        ],
    )(_make_sc_body(k, first=(k == 0)))


_sc_calls = [_make_sc(k) for k in range(NSPLIT)]


def _final_body(ni_ref, na_ref, part_ref, wsc_ref, w2_ref, out_ref):
    agg = part_ref[0:N, :] + part_ref[NP:NP + N, :]
    s = jnp.dot(ni_ref[...], wsc_ref[...], preferred_element_type=jnp.float32)
    xo = jnp.dot(agg, w2_ref[...], preferred_element_type=jnp.float32)
    c_s = math.sin(math.pi / 8.0) / math.sqrt(float(D))
    c_x = math.cos(math.pi / 8.0) / math.sqrt(float(D))
    out_ref[...] = (s * c_s + xo * c_x) * na_ref[...]


_final = pl.pallas_call(
    _final_body,
    out_shape=jax.ShapeDtypeStruct((N, D), jnp.float32),
)


def kernel(node_input, node_attr, edge_src, edge_dst, edge_attr,
           edge_length_embedded, W_sc, W_lin1, W_fc1, W_fc2, W_lin2):
    w_slices = [
        _make_edge_fc(k)(edge_length_embedded, edge_attr, W_fc1, W_fc2)
        for k in range(NSPLIT)
    ]
    x = _node_lin(node_input, node_attr, W_lin1[:, 0, :])
    acc = jnp.zeros((NP, D), dtype=jnp.float32)
    for k in range(NSPLIT):
        acc = _sc_calls[k](x, w_slices[k], edge_src, edge_dst, acc)
    return _final(node_input, node_attr, acc, W_sc[:, 0, :], W_lin2[:, 0, :])


# speedup vs baseline: 1.7936x; 1.4893x over previous
import functools
import math

import jax
import jax.numpy as jnp
from jax import lax
from jax.experimental import pallas as pl
from jax.experimental.pallas import tpu as pltpu
from jax.experimental.pallas import tpu_sc as plsc

N = 10000
E = 320000
D = 128
FC0 = 16
FC1 = 64
NUM_NEIGHBORS = 32.0

NC = 2
NS = 16
NW = NC * NS
CH = 40
NP = 10240
RPT = NP // NS

NSPLIT = 5
ES = E // NSPLIT
EPT = ES // NW
NCHUNK = EPT // CH
BE = 6400
NBLK = ES // BE


def _edge_fc_body(elemT_ref, wfc1_ref, wfc2_ref, out_ref):
    h = lax.dot_general(elemT_ref[...], wfc1_ref[...], (((0,), (0,)), ((), ())),
                        preferred_element_type=jnp.float32)
    h = h * (1.0 / math.sqrt(float(FC0)))
    h = h * jax.nn.sigmoid(h)
    w = jnp.dot(h, wfc2_ref[...], preferred_element_type=jnp.float32)
    scale = (1.0 / math.sqrt(float(FC1))) * (1.0 / math.sqrt(NUM_NEIGHBORS))
    out_ref[...] = w * scale


def _make_edge_fc(k):
    return pl.pallas_call(
        _edge_fc_body,
        grid=(NBLK,),
        in_specs=[
            pl.BlockSpec((FC0, BE), lambda i: (0, k * NBLK + i)),
            pl.BlockSpec((FC0, FC1), lambda i: (0, 0)),
            pl.BlockSpec((FC1, D), lambda i: (0, 0)),
        ],
        out_specs=pl.BlockSpec((BE, D), lambda i: (i, 0)),
        out_shape=jax.ShapeDtypeStruct((ES, D), jnp.float32),
    )


def _node_lin_body(ni_ref, na_ref, w1_ref, out_ref):
    x = jnp.dot(ni_ref[...], w1_ref[...], preferred_element_type=jnp.float32)
    out_ref[...] = x * na_ref[...] * (1.0 / math.sqrt(float(D)))


_node_lin = pl.pallas_call(
    _node_lin_body,
    out_shape=jax.ShapeDtypeStruct((N, D), jnp.float32),
)


def _make_sc_body(k, first):
    def _sc_body(x_hbm, w_hbm, ea_hbm, src_hbm, dst_hbm, init_hbm, out_hbm,
                 src0_v, src1_v, sdst0_v, sdst1_v, ea0_v, ea1_v,
                 xr0_v, xr1_v, wr0_v, wr1_v, acc_sh,
                 sem_g0, sem_g1, sem_w0, sem_w1, sem_i0, sem_i1,
                 sem_s0, sem_s1, sem_z):
        c = lax.axis_index("c")
        s = lax.axis_index("s")
        wid = s * NC + c
        row0 = s * RPT
        if first:
            seed = pltpu.async_copy(init_hbm.at[pl.ds(row0, RPT)],
                                    acc_sh.at[pl.ds(row0, RPT)], sem_z)
        else:
            seed = pltpu.async_copy(init_hbm.at[pl.ds(c * NP + row0, RPT)],
                                    acc_sh.at[pl.ds(row0, RPT)], sem_z)

        src = (src0_v, src1_v)
        sdst = (sdst0_v, sdst1_v)
        ea = (ea0_v, ea1_v)
        xr = (xr0_v, xr1_v)
        wr = (wr0_v, wr1_v)
        sem_g = (sem_g0, sem_g1)
        sem_w = (sem_w0, sem_w1)
        sem_i = (sem_i0, sem_i1)
        sem_s = (sem_s0, sem_s1)
        ibase = k * ES + wid * EPT
        wbase = wid * EPT

        def idx_start(i, b):
            @pl.when(i < NCHUNK)
            def _():
                base = ibase + jnp.minimum(i, NCHUNK - 1) * CH
                pltpu.async_copy(src_hbm.at[pl.ds(base, CH)], src[b], sem_i[b])

        def idx_wait(i, b):
            @pl.when(i < NCHUNK)
            def _():
                pltpu.make_async_copy(
                    src_hbm.at[pl.ds(0, CH)], src[b], sem_i[b]).wait()

        def data_start(i, b):
            @pl.when(i < NCHUNK)
            def _():
                base = wbase + jnp.minimum(i, NCHUNK - 1) * CH
                dbase = ibase + jnp.minimum(i, NCHUNK - 1) * CH
                pltpu.async_copy(w_hbm.at[pl.ds(base, CH)], wr[b], sem_w[b])
                pltpu.async_copy(dst_hbm.at[pl.ds(dbase, CH)], sdst[b], sem_w[b])
                pltpu.async_copy(ea_hbm.at[pl.ds(dbase, CH)],
                                 ea[b].at[pl.ds(0, CH)], sem_w[b])
                pltpu.async_copy(x_hbm.at[src[b]], xr[b], sem_g[b])

        def data_wait(b):
            pltpu.make_async_copy(w_hbm.at[pl.ds(0, CH)], wr[b], sem_w[b]).wait()
            pltpu.make_async_copy(dst_hbm.at[pl.ds(0, CH)], sdst[b], sem_w[b]).wait()
            pltpu.make_async_copy(ea_hbm.at[pl.ds(0, CH)],
                                  ea[b].at[pl.ds(0, CH)], sem_w[b]).wait()
            pltpu.make_async_copy(x_hbm.at[pl.ds(0, CH)], xr[b], sem_g[b]).wait()

        def scatter_wait(i, b):
            @pl.when(i >= 0)
            def _():
                pltpu.make_async_copy(xr[b], acc_sh.at[sdst[b]], sem_s[b]).wait()

        idx_start(0, 0)
        idx_wait(0, 0)
        data_start(0, 0)
        idx_start(1, 1)
        seed.wait()
        plsc.subcore_barrier()

        def step(i, b):
            scatter_wait(i - 1, 1 - b)
            idx_wait(i + 1, 1 - b)
            data_start(i + 1, 1 - b)
            data_wait(b)
            idx_start(i + 2, b)

            def mrow(r, carry2):
                a = ea[b][pl.ds(r, 16)][0]
                for cc in range(D // 16):
                    sl = pl.ds(cc * 16, 16)
                    xr[b][r, sl] = xr[b][r, sl] * (wr[b][r, sl] * a)
                return carry2

            lax.fori_loop(0, CH, mrow, 0)
            pltpu.async_copy(xr[b], acc_sh.at[sdst[b]], sem_s[b], add=True)

        def pair(j, carry):
            step(2 * j, 0)
            step(2 * j + 1, 1)
            return carry

        lax.fori_loop(0, NCHUNK // 2, pair, 0)
        scatter_wait(NCHUNK - 1, 1)

        plsc.subcore_barrier()
        pltpu.sync_copy(acc_sh.at[pl.ds(row0, RPT)],
                        out_hbm.at[pl.ds(c * NP + row0, RPT)])

    return _sc_body


def _make_sc(k):
    return functools.partial(
        pl.kernel,
        out_type=jax.ShapeDtypeStruct((NC * NP, D), jnp.float32),
        mesh=plsc.VectorSubcoreMesh(core_axis_name="c", subcore_axis_name="s"),
        scratch_types=[
            pltpu.VMEM((CH,), jnp.int32),
            pltpu.VMEM((CH,), jnp.int32),
            pltpu.VMEM((CH,), jnp.int32),
            pltpu.VMEM((CH,), jnp.int32),
            pltpu.VMEM((CH + 16,), jnp.float32),
            pltpu.VMEM((CH + 16,), jnp.float32),
            pltpu.VMEM((CH, D), jnp.float32),
            pltpu.VMEM((CH, D), jnp.float32),
            pltpu.VMEM((CH, D), jnp.float32),
            pltpu.VMEM((CH, D), jnp.float32),
            pltpu.VMEM_SHARED((NP, D), jnp.float32),
            pltpu.SemaphoreType.DMA,
            pltpu.SemaphoreType.DMA,
            pltpu.SemaphoreType.DMA,
            pltpu.SemaphoreType.DMA,
            pltpu.SemaphoreType.DMA,
            pltpu.SemaphoreType.DMA,
            pltpu.SemaphoreType.DMA,
            pltpu.SemaphoreType.DMA,
            pltpu.SemaphoreType.DMA,
        ],
    )(_make_sc_body(k, first=(k == 0)))


_sc_calls = [_make_sc(k) for k in range(NSPLIT)]


def _final_body(ni_ref, na_ref, part_ref, wsc_ref, w2_ref, out_ref):
    agg = part_ref[0:N, :] + part_ref[NP:NP + N, :]
    s = jnp.dot(ni_ref[...], wsc_ref[...], preferred_element_type=jnp.float32)
    xo = jnp.dot(agg, w2_ref[...], preferred_element_type=jnp.float32)
    c_s = math.sin(math.pi / 8.0) / math.sqrt(float(D))
    c_x = math.cos(math.pi / 8.0) / math.sqrt(float(D))
    out_ref[...] = (s * c_s + xo * c_x) * na_ref[...]


_final = pl.pallas_call(
    _final_body,
    out_shape=jax.ShapeDtypeStruct((N, D), jnp.float32),
)


def kernel(node_input, node_attr, edge_src, edge_dst, edge_attr,
           edge_length_embedded, W_sc, W_lin1, W_fc1, W_fc2, W_lin2):
    elemT = edge_length_embedded.T
    ea_flat = edge_attr.reshape(E)
    w_slices = [
        _make_edge_fc(k)(elemT, W_fc1, W_fc2)
        for k in range(NSPLIT)
    ]
    x = _node_lin(node_input, node_attr, W_lin1[:, 0, :])
    acc = jnp.zeros((NP, D), dtype=jnp.float32)
    for k in range(NSPLIT):
        acc = _sc_calls[k](x, w_slices[k], ea_flat, edge_src, edge_dst, acc)
    return _final(node_input, node_attr, acc, W_sc[:, 0, :], W_lin2[:, 0, :])


# speedup vs baseline: 1.9663x; 1.0963x over previous
import functools
import math

import jax
import jax.numpy as jnp
from jax import lax
from jax.experimental import pallas as pl
from jax.experimental.pallas import tpu as pltpu
from jax.experimental.pallas import tpu_sc as plsc

N = 10000
E = 320000
D = 128
FC0 = 16
FC1 = 64
NUM_NEIGHBORS = 32.0

NC = 2
NS = 16
NW = NC * NS
CH = 40
NP = 10240
RPT = NP // NS

NSPLIT = 2
ES = E // NSPLIT
EPT = ES // NW
NCHUNK = EPT // CH
BE = 6400
NBLK = ES // BE


def _edge_fc_body(elemT_ref, wfc1_ref, wfc2_ref, out_ref):
    h = lax.dot_general(elemT_ref[...], wfc1_ref[...], (((0,), (0,)), ((), ())),
                        preferred_element_type=jnp.float32)
    h = h * (1.0 / math.sqrt(float(FC0)))
    h = h * jax.nn.sigmoid(h)
    w = jnp.dot(h, wfc2_ref[...], preferred_element_type=jnp.float32)
    scale = (1.0 / math.sqrt(float(FC1))) * (1.0 / math.sqrt(NUM_NEIGHBORS))
    out_ref[...] = w * scale


def _make_edge_fc(k):
    return pl.pallas_call(
        _edge_fc_body,
        grid=(NBLK,),
        in_specs=[
            pl.BlockSpec((FC0, BE), lambda i: (0, k * NBLK + i)),
            pl.BlockSpec((FC0, FC1), lambda i: (0, 0)),
            pl.BlockSpec((FC1, D), lambda i: (0, 0)),
        ],
        out_specs=pl.BlockSpec((BE, D), lambda i: (i, 0)),
        out_shape=jax.ShapeDtypeStruct((ES, D), jnp.float32),
    )


def _node_lin_body(ni_ref, na_ref, w1_ref, out_ref):
    x = jnp.dot(ni_ref[...], w1_ref[...], preferred_element_type=jnp.float32)
    out_ref[...] = x * na_ref[...] * (1.0 / math.sqrt(float(D)))


_node_lin = pl.pallas_call(
    _node_lin_body,
    out_shape=jax.ShapeDtypeStruct((N, D), jnp.float32),
)


def _make_sc_body(k, first):
    def _sc_body(x_hbm, w_hbm, ea_hbm, src_hbm, dst_hbm, init_hbm, out_hbm,
                 src0_v, src1_v, sdst0_v, sdst1_v, ea0_v, ea1_v,
                 xr0_v, xr1_v, wr0_v, wr1_v, acc_sh,
                 sem_g0, sem_g1, sem_w0, sem_w1, sem_i0, sem_i1,
                 sem_s0, sem_s1, sem_z):
        c = lax.axis_index("c")
        s = lax.axis_index("s")
        wid = s * NC + c
        row0 = s * RPT
        if first:
            seed = pltpu.async_copy(init_hbm.at[pl.ds(row0, RPT)],
                                    acc_sh.at[pl.ds(row0, RPT)], sem_z)
        else:
            seed = pltpu.async_copy(init_hbm.at[pl.ds(c * NP + row0, RPT)],
                                    acc_sh.at[pl.ds(row0, RPT)], sem_z)

        src = (src0_v, src1_v)
        sdst = (sdst0_v, sdst1_v)
        ea = (ea0_v, ea1_v)
        xr = (xr0_v, xr1_v)
        wr = (wr0_v, wr1_v)
        sem_g = (sem_g0, sem_g1)
        sem_w = (sem_w0, sem_w1)
        sem_i = (sem_i0, sem_i1)
        sem_s = (sem_s0, sem_s1)
        ibase = k * ES + wid * EPT
        wbase = wid * EPT

        def idx_start(i, b):
            @pl.when(i < NCHUNK)
            def _():
                base = ibase + jnp.minimum(i, NCHUNK - 1) * CH
                pltpu.async_copy(src_hbm.at[pl.ds(base, CH)], src[b], sem_i[b])

        def idx_wait(i, b):
            @pl.when(i < NCHUNK)
            def _():
                pltpu.make_async_copy(
                    src_hbm.at[pl.ds(0, CH)], src[b], sem_i[b]).wait()

        def data_start(i, b):
            @pl.when(i < NCHUNK)
            def _():
                base = wbase + jnp.minimum(i, NCHUNK - 1) * CH
                dbase = ibase + jnp.minimum(i, NCHUNK - 1) * CH
                pltpu.async_copy(w_hbm.at[pl.ds(base, CH)], wr[b], sem_w[b])
                pltpu.async_copy(dst_hbm.at[pl.ds(dbase, CH)], sdst[b], sem_w[b])
                pltpu.async_copy(ea_hbm.at[pl.ds(dbase, CH)],
                                 ea[b].at[pl.ds(0, CH)], sem_w[b])
                pltpu.async_copy(x_hbm.at[src[b]], xr[b], sem_g[b])

        def data_wait(b):
            pltpu.make_async_copy(w_hbm.at[pl.ds(0, CH)], wr[b], sem_w[b]).wait()
            pltpu.make_async_copy(dst_hbm.at[pl.ds(0, CH)], sdst[b], sem_w[b]).wait()
            pltpu.make_async_copy(ea_hbm.at[pl.ds(0, CH)],
                                  ea[b].at[pl.ds(0, CH)], sem_w[b]).wait()
            pltpu.make_async_copy(x_hbm.at[pl.ds(0, CH)], xr[b], sem_g[b]).wait()

        def scatter_wait(i, b):
            @pl.when(i >= 0)
            def _():
                pltpu.make_async_copy(xr[b], acc_sh.at[sdst[b]], sem_s[b]).wait()

        idx_start(0, 0)
        idx_wait(0, 0)
        data_start(0, 0)
        idx_start(1, 1)
        seed.wait()
        plsc.subcore_barrier()

        def step(i, b):
            scatter_wait(i - 1, 1 - b)
            idx_wait(i + 1, 1 - b)
            data_start(i + 1, 1 - b)
            data_wait(b)
            idx_start(i + 2, b)

            def mrow(r, carry2):
                a = ea[b][pl.ds(r, 16)][0]
                for cc in range(D // 16):
                    sl = pl.ds(cc * 16, 16)
                    xr[b][r, sl] = xr[b][r, sl] * (wr[b][r, sl] * a)
                return carry2

            lax.fori_loop(0, CH, mrow, 0)
            pltpu.async_copy(xr[b], acc_sh.at[sdst[b]], sem_s[b], add=True)

        def pair(j, carry):
            step(2 * j, 0)
            step(2 * j + 1, 1)
            return carry

        lax.fori_loop(0, NCHUNK // 2, pair, 0)
        if NCHUNK % 2:
            step(NCHUNK - 1, 0)
        scatter_wait(NCHUNK - 1, (NCHUNK - 1) % 2)

        plsc.subcore_barrier()
        pltpu.sync_copy(acc_sh.at[pl.ds(row0, RPT)],
                        out_hbm.at[pl.ds(c * NP + row0, RPT)])

    return _sc_body


def _make_sc(k):
    return functools.partial(
        pl.kernel,
        out_type=jax.ShapeDtypeStruct((NC * NP, D), jnp.float32),
        mesh=plsc.VectorSubcoreMesh(core_axis_name="c", subcore_axis_name="s"),
        scratch_types=[
            pltpu.VMEM((CH,), jnp.int32),
            pltpu.VMEM((CH,), jnp.int32),
            pltpu.VMEM((CH,), jnp.int32),
            pltpu.VMEM((CH,), jnp.int32),
            pltpu.VMEM((CH + 16,), jnp.float32),
            pltpu.VMEM((CH + 16,), jnp.float32),
            pltpu.VMEM((CH, D), jnp.float32),
            pltpu.VMEM((CH, D), jnp.float32),
            pltpu.VMEM((CH, D), jnp.float32),
            pltpu.VMEM((CH, D), jnp.float32),
            pltpu.VMEM_SHARED((NP, D), jnp.float32),
            pltpu.SemaphoreType.DMA,
            pltpu.SemaphoreType.DMA,
            pltpu.SemaphoreType.DMA,
            pltpu.SemaphoreType.DMA,
            pltpu.SemaphoreType.DMA,
            pltpu.SemaphoreType.DMA,
            pltpu.SemaphoreType.DMA,
            pltpu.SemaphoreType.DMA,
            pltpu.SemaphoreType.DMA,
        ],
    )(_make_sc_body(k, first=(k == 0)))


_sc_calls = [_make_sc(k) for k in range(NSPLIT)]


def _final_body(ni_ref, na_ref, part_ref, wsc_ref, w2_ref, out_ref):
    agg = part_ref[0:N, :] + part_ref[NP:NP + N, :]
    s = jnp.dot(ni_ref[...], wsc_ref[...], preferred_element_type=jnp.float32)
    xo = jnp.dot(agg, w2_ref[...], preferred_element_type=jnp.float32)
    c_s = math.sin(math.pi / 8.0) / math.sqrt(float(D))
    c_x = math.cos(math.pi / 8.0) / math.sqrt(float(D))
    out_ref[...] = (s * c_s + xo * c_x) * na_ref[...]


_final = pl.pallas_call(
    _final_body,
    out_shape=jax.ShapeDtypeStruct((N, D), jnp.float32),
)


def kernel(node_input, node_attr, edge_src, edge_dst, edge_attr,
           edge_length_embedded, W_sc, W_lin1, W_fc1, W_fc2, W_lin2):
    elemT = edge_length_embedded.T
    ea_flat = edge_attr.reshape(E)
    w_slices = [
        _make_edge_fc(k)(elemT, W_fc1, W_fc2)
        for k in range(NSPLIT)
    ]
    x = _node_lin(node_input, node_attr, W_lin1[:, 0, :])
    acc = jnp.zeros((NP, D), dtype=jnp.float32)
    for k in range(NSPLIT):
        acc = _sc_calls[k](x, w_slices[k], ea_flat, edge_src, edge_dst, acc)
    return _final(node_input, node_attr, acc, W_sc[:, 0, :], W_lin2[:, 0, :])


# speedup vs baseline: 2.0011x; 1.0177x over previous
import functools
import math

import jax
import jax.numpy as jnp
from jax import lax
from jax.experimental import pallas as pl
from jax.experimental.pallas import tpu as pltpu
from jax.experimental.pallas import tpu_sc as plsc

N = 10000
E = 320000
D = 128
FC0 = 16
FC1 = 64
NUM_NEIGHBORS = 32.0

NC = 2
NS = 16
NW = NC * NS
CH = 40
NP = 10240
RPT = NP // NS

SPLITS = (25600, 147200, 147200)
NSPLIT = len(SPLITS)
BE = 6400


def _edge_fc_body(elemT_ref, wfc1_ref, wfc2_ref, out_ref):
    h = lax.dot_general(elemT_ref[...], wfc1_ref[...], (((0,), (0,)), ((), ())),
                        preferred_element_type=jnp.float32)
    h = h * (1.0 / math.sqrt(float(FC0)))
    h = h * jax.nn.sigmoid(h)
    w = jnp.dot(h, wfc2_ref[...], preferred_element_type=jnp.float32)
    scale = (1.0 / math.sqrt(float(FC1))) * (1.0 / math.sqrt(NUM_NEIGHBORS))
    out_ref[...] = w * scale


def _make_edge_fc(k):
    es = SPLITS[k]
    blk0 = sum(SPLITS[:k]) // BE
    return pl.pallas_call(
        _edge_fc_body,
        grid=(es // BE,),
        in_specs=[
            pl.BlockSpec((FC0, BE), lambda i: (0, blk0 + i)),
            pl.BlockSpec((FC0, FC1), lambda i: (0, 0)),
            pl.BlockSpec((FC1, D), lambda i: (0, 0)),
        ],
        out_specs=pl.BlockSpec((BE, D), lambda i: (i, 0)),
        out_shape=jax.ShapeDtypeStruct((es, D), jnp.float32),
    )


def _node_lin_body(ni_ref, na_ref, w1_ref, out_ref):
    x = jnp.dot(ni_ref[...], w1_ref[...], preferred_element_type=jnp.float32)
    out_ref[...] = x * na_ref[...] * (1.0 / math.sqrt(float(D)))


_node_lin = pl.pallas_call(
    _node_lin_body,
    out_shape=jax.ShapeDtypeStruct((N, D), jnp.float32),
)


def _make_sc_body(k, first):
    ES = SPLITS[k]
    E0 = sum(SPLITS[:k])
    EPT = ES // NW
    NCHUNK = EPT // CH
    def _sc_body(x_hbm, w_hbm, ea_hbm, src_hbm, dst_hbm, init_hbm, out_hbm,
                 src0_v, src1_v, sdst0_v, sdst1_v, ea0_v, ea1_v,
                 xr0_v, xr1_v, wr0_v, wr1_v, acc_sh,
                 sem_g0, sem_g1, sem_w0, sem_w1, sem_i0, sem_i1,
                 sem_s0, sem_s1, sem_z):
        c = lax.axis_index("c")
        s = lax.axis_index("s")
        wid = s * NC + c
        row0 = s * RPT
        if first:
            seed = pltpu.async_copy(init_hbm.at[pl.ds(row0, RPT)],
                                    acc_sh.at[pl.ds(row0, RPT)], sem_z)
        else:
            seed = pltpu.async_copy(init_hbm.at[pl.ds(c * NP + row0, RPT)],
                                    acc_sh.at[pl.ds(row0, RPT)], sem_z)

        src = (src0_v, src1_v)
        sdst = (sdst0_v, sdst1_v)
        ea = (ea0_v, ea1_v)
        xr = (xr0_v, xr1_v)
        wr = (wr0_v, wr1_v)
        sem_g = (sem_g0, sem_g1)
        sem_w = (sem_w0, sem_w1)
        sem_i = (sem_i0, sem_i1)
        sem_s = (sem_s0, sem_s1)
        ibase = E0 + wid * EPT
        wbase = wid * EPT

        def idx_start(i, b):
            @pl.when(i < NCHUNK)
            def _():
                base = ibase + jnp.minimum(i, NCHUNK - 1) * CH
                pltpu.async_copy(src_hbm.at[pl.ds(base, CH)], src[b], sem_i[b])

        def idx_wait(i, b):
            @pl.when(i < NCHUNK)
            def _():
                pltpu.make_async_copy(
                    src_hbm.at[pl.ds(0, CH)], src[b], sem_i[b]).wait()

        def data_start(i, b):
            @pl.when(i < NCHUNK)
            def _():
                base = wbase + jnp.minimum(i, NCHUNK - 1) * CH
                dbase = ibase + jnp.minimum(i, NCHUNK - 1) * CH
                pltpu.async_copy(w_hbm.at[pl.ds(base, CH)], wr[b], sem_w[b])
                pltpu.async_copy(dst_hbm.at[pl.ds(dbase, CH)], sdst[b], sem_w[b])
                pltpu.async_copy(ea_hbm.at[pl.ds(dbase, CH)],
                                 ea[b].at[pl.ds(0, CH)], sem_w[b])
                pltpu.async_copy(x_hbm.at[src[b]], xr[b], sem_g[b])

        def data_wait(b):
            pltpu.make_async_copy(w_hbm.at[pl.ds(0, CH)], wr[b], sem_w[b]).wait()
            pltpu.make_async_copy(dst_hbm.at[pl.ds(0, CH)], sdst[b], sem_w[b]).wait()
            pltpu.make_async_copy(ea_hbm.at[pl.ds(0, CH)],
                                  ea[b].at[pl.ds(0, CH)], sem_w[b]).wait()
            pltpu.make_async_copy(x_hbm.at[pl.ds(0, CH)], xr[b], sem_g[b]).wait()

        def scatter_wait(i, b):
            @pl.when(i >= 0)
            def _():
                pltpu.make_async_copy(xr[b], acc_sh.at[sdst[b]], sem_s[b]).wait()

        idx_start(0, 0)
        idx_wait(0, 0)
        data_start(0, 0)
        idx_start(1, 1)
        seed.wait()
        plsc.subcore_barrier()

        def step(i, b):
            scatter_wait(i - 1, 1 - b)
            idx_wait(i + 1, 1 - b)
            data_start(i + 1, 1 - b)
            data_wait(b)
            idx_start(i + 2, b)

            def mrow(r, carry2):
                a = ea[b][pl.ds(r, 16)][0]
                for cc in range(D // 16):
                    sl = pl.ds(cc * 16, 16)
                    xr[b][r, sl] = xr[b][r, sl] * (wr[b][r, sl] * a)
                return carry2

            lax.fori_loop(0, CH, mrow, 0)
            pltpu.async_copy(xr[b], acc_sh.at[sdst[b]], sem_s[b], add=True)

        def pair(j, carry):
            step(2 * j, 0)
            step(2 * j + 1, 1)
            return carry

        lax.fori_loop(0, NCHUNK // 2, pair, 0)
        if NCHUNK % 2:
            step(NCHUNK - 1, 0)
        scatter_wait(NCHUNK - 1, (NCHUNK - 1) % 2)

        plsc.subcore_barrier()
        pltpu.sync_copy(acc_sh.at[pl.ds(row0, RPT)],
                        out_hbm.at[pl.ds(c * NP + row0, RPT)])

    return _sc_body


def _make_sc(k):
    return functools.partial(
        pl.kernel,
        out_type=jax.ShapeDtypeStruct((NC * NP, D), jnp.float32),
        mesh=plsc.VectorSubcoreMesh(core_axis_name="c", subcore_axis_name="s"),
        scratch_types=[
            pltpu.VMEM((CH,), jnp.int32),
            pltpu.VMEM((CH,), jnp.int32),
            pltpu.VMEM((CH,), jnp.int32),
            pltpu.VMEM((CH,), jnp.int32),
            pltpu.VMEM((CH + 16,), jnp.float32),
            pltpu.VMEM((CH + 16,), jnp.float32),
            pltpu.VMEM((CH, D), jnp.float32),
            pltpu.VMEM((CH, D), jnp.float32),
            pltpu.VMEM((CH, D), jnp.float32),
            pltpu.VMEM((CH, D), jnp.float32),
            pltpu.VMEM_SHARED((NP, D), jnp.float32),
            pltpu.SemaphoreType.DMA,
            pltpu.SemaphoreType.DMA,
            pltpu.SemaphoreType.DMA,
            pltpu.SemaphoreType.DMA,
            pltpu.SemaphoreType.DMA,
            pltpu.SemaphoreType.DMA,
            pltpu.SemaphoreType.DMA,
            pltpu.SemaphoreType.DMA,
            pltpu.SemaphoreType.DMA,
        ],
    )(_make_sc_body(k, first=(k == 0)))


_sc_calls = [_make_sc(k) for k in range(NSPLIT)]


def _final_body(ni_ref, na_ref, part_ref, wsc_ref, w2_ref, out_ref):
    agg = part_ref[0:N, :] + part_ref[NP:NP + N, :]
    s = jnp.dot(ni_ref[...], wsc_ref[...], preferred_element_type=jnp.float32)
    xo = jnp.dot(agg, w2_ref[...], preferred_element_type=jnp.float32)
    c_s = math.sin(math.pi / 8.0) / math.sqrt(float(D))
    c_x = math.cos(math.pi / 8.0) / math.sqrt(float(D))
    out_ref[...] = (s * c_s + xo * c_x) * na_ref[...]


_final = pl.pallas_call(
    _final_body,
    out_shape=jax.ShapeDtypeStruct((N, D), jnp.float32),
)


def kernel(node_input, node_attr, edge_src, edge_dst, edge_attr,
           edge_length_embedded, W_sc, W_lin1, W_fc1, W_fc2, W_lin2):
    elemT = edge_length_embedded.T
    ea_flat = edge_attr.reshape(E)
    w_slices = [
        _make_edge_fc(k)(elemT, W_fc1, W_fc2)
        for k in range(NSPLIT)
    ]
    x = _node_lin(node_input, node_attr, W_lin1[:, 0, :])
    acc = jnp.zeros((NP, D), dtype=jnp.float32)
    for k in range(NSPLIT):
        acc = _sc_calls[k](x, w_slices[k], ea_flat, edge_src, edge_dst, acc)
    return _final(node_input, node_attr, acc, W_sc[:, 0, :], W_lin2[:, 0, :])


# speedup vs baseline: 2.0608x; 1.0299x over previous
import functools
import math

import jax
import jax.numpy as jnp
from jax import lax
from jax.experimental import pallas as pl
from jax.experimental.pallas import tpu as pltpu
from jax.experimental.pallas import tpu_sc as plsc

N = 10000
E = 320000
D = 128
FC0 = 16
FC1 = 64
NUM_NEIGHBORS = 32.0

NC = 2
NS = 16
NW = NC * NS
CH = 40
NP = 10240
RPT = NP // NS

SPLITS = (25600, 147200, 147200)
NSPLIT = len(SPLITS)
BE = 6400


def _edge_fc_body(elemT_ref, wfc1_ref, wfc2_ref, out_ref):
    h = lax.dot_general(elemT_ref[...], wfc1_ref[...], (((0,), (0,)), ((), ())),
                        preferred_element_type=jnp.float32)
    h = h * (1.0 / math.sqrt(float(FC0)))
    h = h * jax.nn.sigmoid(h)
    w = jnp.dot(h, wfc2_ref[...], preferred_element_type=jnp.float32)
    scale = (1.0 / math.sqrt(float(FC1))) * (1.0 / math.sqrt(NUM_NEIGHBORS))
    out_ref[...] = w * scale


def _make_edge_fc(k):
    es = SPLITS[k]
    blk0 = sum(SPLITS[:k]) // BE
    return pl.pallas_call(
        _edge_fc_body,
        grid=(es // BE,),
        in_specs=[
            pl.BlockSpec((FC0, BE), lambda i: (0, blk0 + i)),
            pl.BlockSpec((FC0, FC1), lambda i: (0, 0)),
            pl.BlockSpec((FC1, D), lambda i: (0, 0)),
        ],
        out_specs=pl.BlockSpec((BE, D), lambda i: (i, 0)),
        out_shape=jax.ShapeDtypeStruct((es, D), jnp.float32),
    )


def _ea_flat_body(eaT_ref, out_ref):
    out_ref[...] = eaT_ref[0, :]


_ea_flatten = pl.pallas_call(
    _ea_flat_body,
    out_shape=jax.ShapeDtypeStruct((E,), jnp.float32),
)


def _node_lin_body(ni_ref, na_ref, w1_ref, out_ref):
    x = jnp.dot(ni_ref[...], w1_ref[...], preferred_element_type=jnp.float32)
    out_ref[...] = x * na_ref[...] * (1.0 / math.sqrt(float(D)))


_node_lin = pl.pallas_call(
    _node_lin_body,
    out_shape=jax.ShapeDtypeStruct((N, D), jnp.float32),
)


def _make_sc_body(k, first):
    ES = SPLITS[k]
    E0 = sum(SPLITS[:k])
    EPT = ES // NW
    NCHUNK = EPT // CH
    def _sc_body(x_hbm, w_hbm, ea_hbm, src_hbm, dst_hbm, init_hbm, out_hbm,
                 src0_v, src1_v, sdst0_v, sdst1_v, ea0_v, ea1_v,
                 xr0_v, xr1_v, wr0_v, wr1_v, acc_sh,
                 sem_g0, sem_g1, sem_w0, sem_w1, sem_i0, sem_i1,
                 sem_s0, sem_s1, sem_z):
        c = lax.axis_index("c")
        s = lax.axis_index("s")
        wid = s * NC + c
        row0 = s * RPT
        if first:
            seed = pltpu.async_copy(init_hbm.at[pl.ds(row0, RPT)],
                                    acc_sh.at[pl.ds(row0, RPT)], sem_z)
        else:
            seed = pltpu.async_copy(init_hbm.at[pl.ds(c * NP + row0, RPT)],
                                    acc_sh.at[pl.ds(row0, RPT)], sem_z)

        src = (src0_v, src1_v)
        sdst = (sdst0_v, sdst1_v)
        ea = (ea0_v, ea1_v)
        xr = (xr0_v, xr1_v)
        wr = (wr0_v, wr1_v)
        sem_g = (sem_g0, sem_g1)
        sem_w = (sem_w0, sem_w1)
        sem_i = (sem_i0, sem_i1)
        sem_s = (sem_s0, sem_s1)
        ibase = E0 + wid * EPT
        wbase = wid * EPT

        def idx_start(i, b):
            @pl.when(i < NCHUNK)
            def _():
                base = ibase + jnp.minimum(i, NCHUNK - 1) * CH
                pltpu.async_copy(src_hbm.at[pl.ds(base, CH)], src[b], sem_i[b])

        def idx_wait(i, b):
            @pl.when(i < NCHUNK)
            def _():
                pltpu.make_async_copy(
                    src_hbm.at[pl.ds(0, CH)], src[b], sem_i[b]).wait()

        def data_start(i, b):
            @pl.when(i < NCHUNK)
            def _():
                base = wbase + jnp.minimum(i, NCHUNK - 1) * CH
                dbase = ibase + jnp.minimum(i, NCHUNK - 1) * CH
                pltpu.async_copy(w_hbm.at[pl.ds(base, CH)], wr[b], sem_w[b])
                pltpu.async_copy(dst_hbm.at[pl.ds(dbase, CH)], sdst[b], sem_w[b])
                pltpu.async_copy(ea_hbm.at[pl.ds(dbase, CH)],
                                 ea[b].at[pl.ds(0, CH)], sem_w[b])
                pltpu.async_copy(x_hbm.at[src[b]], xr[b], sem_g[b])

        def data_wait(b):
            pltpu.make_async_copy(w_hbm.at[pl.ds(0, CH)], wr[b], sem_w[b]).wait()
            pltpu.make_async_copy(dst_hbm.at[pl.ds(0, CH)], sdst[b], sem_w[b]).wait()
            pltpu.make_async_copy(ea_hbm.at[pl.ds(0, CH)],
                                  ea[b].at[pl.ds(0, CH)], sem_w[b]).wait()
            pltpu.make_async_copy(x_hbm.at[pl.ds(0, CH)], xr[b], sem_g[b]).wait()

        def scatter_wait(i, b):
            @pl.when(i >= 0)
            def _():
                pltpu.make_async_copy(xr[b], acc_sh.at[sdst[b]], sem_s[b]).wait()

        idx_start(0, 0)
        idx_wait(0, 0)
        data_start(0, 0)
        idx_start(1, 1)
        seed.wait()
        plsc.subcore_barrier()

        def step(i, b):
            scatter_wait(i - 1, 1 - b)
            idx_wait(i + 1, 1 - b)
            data_start(i + 1, 1 - b)
            data_wait(b)
            idx_start(i + 2, b)

            def mrow(r, carry2):
                a = ea[b][pl.ds(r, 16)][0]
                for cc in range(D // 16):
                    sl = pl.ds(cc * 16, 16)
                    xr[b][r, sl] = xr[b][r, sl] * (wr[b][r, sl] * a)
                return carry2

            lax.fori_loop(0, CH, mrow, 0)
            pltpu.async_copy(xr[b], acc_sh.at[sdst[b]], sem_s[b], add=True)

        def pair(j, carry):
            step(2 * j, 0)
            step(2 * j + 1, 1)
            return carry

        lax.fori_loop(0, NCHUNK // 2, pair, 0)
        if NCHUNK % 2:
            step(NCHUNK - 1, 0)
        scatter_wait(NCHUNK - 1, (NCHUNK - 1) % 2)

        plsc.subcore_barrier()
        pltpu.sync_copy(acc_sh.at[pl.ds(row0, RPT)],
                        out_hbm.at[pl.ds(c * NP + row0, RPT)])

    return _sc_body


def _make_sc(k):
    return functools.partial(
        pl.kernel,
        out_type=jax.ShapeDtypeStruct((NC * NP, D), jnp.float32),
        mesh=plsc.VectorSubcoreMesh(core_axis_name="c", subcore_axis_name="s"),
        scratch_types=[
            pltpu.VMEM((CH,), jnp.int32),
            pltpu.VMEM((CH,), jnp.int32),
            pltpu.VMEM((CH,), jnp.int32),
            pltpu.VMEM((CH,), jnp.int32),
            pltpu.VMEM((CH + 16,), jnp.float32),
            pltpu.VMEM((CH + 16,), jnp.float32),
            pltpu.VMEM((CH, D), jnp.float32),
            pltpu.VMEM((CH, D), jnp.float32),
            pltpu.VMEM((CH, D), jnp.float32),
            pltpu.VMEM((CH, D), jnp.float32),
            pltpu.VMEM_SHARED((NP, D), jnp.float32),
            pltpu.SemaphoreType.DMA,
            pltpu.SemaphoreType.DMA,
            pltpu.SemaphoreType.DMA,
            pltpu.SemaphoreType.DMA,
            pltpu.SemaphoreType.DMA,
            pltpu.SemaphoreType.DMA,
            pltpu.SemaphoreType.DMA,
            pltpu.SemaphoreType.DMA,
            pltpu.SemaphoreType.DMA,
        ],
    )(_make_sc_body(k, first=(k == 0)))


_sc_calls = [_make_sc(k) for k in range(NSPLIT)]


def _final_body(ni_ref, na_ref, part_ref, wsc_ref, w2_ref, out_ref):
    agg = part_ref[0:N, :] + part_ref[NP:NP + N, :]
    s = jnp.dot(ni_ref[...], wsc_ref[...], preferred_element_type=jnp.float32)
    xo = jnp.dot(agg, w2_ref[...], preferred_element_type=jnp.float32)
    c_s = math.sin(math.pi / 8.0) / math.sqrt(float(D))
    c_x = math.cos(math.pi / 8.0) / math.sqrt(float(D))
    out_ref[...] = (s * c_s + xo * c_x) * na_ref[...]


_final = pl.pallas_call(
    _final_body,
    out_shape=jax.ShapeDtypeStruct((N, D), jnp.float32),
)


def kernel(node_input, node_attr, edge_src, edge_dst, edge_attr,
           edge_length_embedded, W_sc, W_lin1, W_fc1, W_fc2, W_lin2):
    elemT = edge_length_embedded.T
    ea_flat = _ea_flatten(edge_attr.T)
    w_slices = [
        _make_edge_fc(k)(elemT, W_fc1, W_fc2)
        for k in range(NSPLIT)
    ]
    x = _node_lin(node_input, node_attr, W_lin1[:, 0, :])
    acc = jnp.zeros((NP, D), dtype=jnp.float32)
    for k in range(NSPLIT):
        acc = _sc_calls[k](x, w_slices[k], ea_flat, edge_src, edge_dst, acc)
    return _final(node_input, node_attr, acc, W_sc[:, 0, :], W_lin2[:, 0, :])
